# baseline (speedup 1.0000x reference)
import functools

import jax
import jax.numpy as jnp
from jax import lax
from jax.experimental import pallas as pl
from jax.experimental.pallas import tpu as pltpu
from jax.experimental.pallas import tpu_sc as plsc

BLK = 256
MAXB = 24
PADROWS = MAXB * BLK
NW = 32
ACHUNK = 4
CW = 32
LANES = 16
WPAD = 128


def _route_body(x_ref, wr_ref, tril_ref, dest_ref, w_ref, blk_ref):
    T = x_ref.shape[0]
    E = wr_ref.shape[0]
    x = x_ref[...]
    logits = lax.dot_general(
        x, wr_ref[...], (((1,), (1,)), ((), ())),
        preferred_element_type=jnp.float32)
    eio = lax.broadcasted_iota(jnp.int32, (T, E), 1)
    m1 = jnp.max(logits, axis=-1, keepdims=True)
    idx1 = jnp.min(jnp.where(logits == m1, eio, E), axis=-1, keepdims=True)
    masked = jnp.where(eio == idx1, -1e30, logits)
    m2 = jnp.max(masked, axis=-1, keepdims=True)
    idx2 = jnp.min(jnp.where(masked == m2, eio, E), axis=-1, keepdims=True)
    e2 = jnp.exp(m2 - m1)
    w1 = 1.0 / (1.0 + e2)
    w2 = e2 / (1.0 + e2)
    sel = ((eio == idx1) | (eio == idx2)).astype(jnp.float32)

    pos = lax.dot_general(
        tril_ref[...], sel.astype(jnp.bfloat16), (((1,), (0,)), ((), ())),
        preferred_element_type=jnp.float32)
    ones_row = jnp.ones((1, T), jnp.float32)
    counts = lax.dot_general(
        ones_row, sel, (((1,), (0,)), ((), ())),
        preferred_element_type=jnp.float32)
    padded = jnp.ceil(counts / BLK) * BLK
    r8 = lax.broadcasted_iota(jnp.int32, (E, E), 0)
    c8 = lax.broadcasted_iota(jnp.int32, (E, E), 1)
    upper = (r8 < c8).astype(jnp.float32)
    offs = lax.dot_general(
        padded, upper, (((1,), (0,)), ((), ())),
        preferred_element_type=jnp.float32)
    basep = offs + pos
    d1 = jnp.sum(jnp.where(eio == idx1, basep, 0.0), axis=-1, keepdims=True)
    d2 = jnp.sum(jnp.where(eio == idx2, basep, 0.0), axis=-1, keepdims=True)
    dest_ref[...] = jnp.concatenate([d1, d2], axis=1).astype(jnp.int32)
    w_ref[...] = jnp.concatenate([w1, w2], axis=1)

    blk_ref[0:1, :] = (offs / BLK).astype(jnp.int32)
    blk_ref[1:2, :] = ((offs + padded) / BLK).astype(jnp.int32)


def _route(x, W_router, tril):
    T = x.shape[0]
    E = W_router.shape[0]
    return pl.pallas_call(
        _route_body,
        out_shape=[
            jax.ShapeDtypeStruct((T, 2), jnp.int32),
            jax.ShapeDtypeStruct((T, 2), jnp.float32),
            jax.ShapeDtypeStruct((2, E), jnp.int32),
        ],
    )(x, W_router, tril)


NF = 4


def _ffn_body(st_ref, en_ref, xs_hbm, wg_ref, wu_ref, wd_ref,
              out_hbm, xs_v, out_v, wg16, wu16, wd16, sem, *, E):
    e = pl.program_id(0)
    f = pl.program_id(1)

    @pl.when((e == 0) & (f == 0))
    def _():
        pltpu.make_async_copy(xs_hbm, xs_v, sem).start()
        pltpu.make_async_copy(xs_hbm, xs_v, sem).wait()

    wg16[...] = wg_ref[0].astype(jnp.bfloat16)
    wu16[...] = wu_ref[0].astype(jnp.bfloat16)
    wd16[...] = wd_ref[0].astype(jnp.bfloat16)

    def body(b, carry):
        r0 = pl.multiple_of(b * BLK, BLK)
        sl = pl.ds(r0, BLK)
        xb = xs_v[sl, :]
        xg = lax.dot_general(
            xb, wg16[...], (((1,), (1,)), ((), ())),
            preferred_element_type=jnp.float32)
        xu = lax.dot_general(
            xb, wu16[...], (((1,), (1,)), ((), ())),
            preferred_element_type=jnp.float32)
        h = ((xg / (1.0 + jnp.exp(-xg))) * xu).astype(jnp.bfloat16)
        part = lax.dot_general(
            h, wd16[...], (((1,), (1,)), ((), ())),
            preferred_element_type=jnp.float32)
        out_v[sl, :] = jnp.where(f == 0, part, out_v[sl, :] + part)
        return carry

    lax.fori_loop(st_ref[e], en_ref[e], body, 0)

    @pl.when((e == E - 1) & (f == NF - 1))
    def _():
        pltpu.make_async_copy(out_v, out_hbm, sem).start()
        pltpu.make_async_copy(out_v, out_hbm, sem).wait()


def _ffn(starts, ends, xs, Wg, Wu, Wd):
    E, F, D = Wg.shape
    FT = F // NF
    return pl.pallas_call(
        functools.partial(_ffn_body, E=E),
        grid=(E, NF),
        in_specs=[
            pl.BlockSpec(memory_space=pltpu.SMEM),
            pl.BlockSpec(memory_space=pltpu.SMEM),
            pl.BlockSpec(memory_space=pl.ANY),
            pl.BlockSpec((1, FT, D), lambda e, f: (e, f, 0)),
            pl.BlockSpec((1, FT, D), lambda e, f: (e, f, 0)),
            pl.BlockSpec((1, D, FT), lambda e, f: (e, 0, f)),
        ],
        out_specs=pl.BlockSpec(memory_space=pl.ANY),
        scratch_shapes=[
            pltpu.VMEM((PADROWS, D), jnp.bfloat16),
            pltpu.VMEM((PADROWS, D), jnp.float32),
            pltpu.VMEM((FT, D), jnp.bfloat16),
            pltpu.VMEM((FT, D), jnp.bfloat16),
            pltpu.VMEM((D, FT), jnp.bfloat16),
            pltpu.SemaphoreType.DMA,
        ],
        out_shape=jax.ShapeDtypeStruct((PADROWS, D), jnp.float32),
        compiler_params=pltpu.CompilerParams(
            vmem_limit_bytes=62 * 1024 * 1024),
    )(starts, ends, xs, Wg, Wu, Wd)


def _dispatch_sc(x, tok3, dest3, w16):
    T, D = x.shape
    APW = ACHUNK * CW
    info = plsc.get_sparse_core_info()
    NC = info.num_cores
    mesh = plsc.VectorSubcoreMesh(core_axis_name="c", subcore_axis_name="s")

    @functools.partial(
        pl.kernel, mesh=mesh,
        out_type=[
            jax.ShapeDtypeStruct((PADROWS, D), jnp.int32),
            jax.ShapeDtypeStruct((PADROWS, WPAD), jnp.float32),
        ],
        scratch_types=[
            pltpu.VMEM((ACHUNK, CW), jnp.int32),
            pltpu.VMEM((ACHUNK, CW), jnp.int32),
            pltpu.VMEM((CW, D), jnp.int32),
            pltpu.VMEM((CW, WPAD), jnp.float32),
            pltpu.SemaphoreType.DMA,
        ],
    )
    def k(x_hbm, tok_hbm, dest_hbm, w16_hbm, xs_hbm, ws_hbm,
          tok_v, dest_v, buf, wbuf, sem):
        wid = lax.axis_index("s") * NC + lax.axis_index("c")
        pltpu.sync_copy(tok_hbm.at[wid], tok_v)
        pltpu.sync_copy(dest_hbm.at[wid], dest_v)
        for ci in range(ACHUNK):
            pltpu.sync_copy(w16_hbm.at[pl.ds(wid * APW + ci * CW, CW)], wbuf)
            pltpu.async_copy(x_hbm.at[tok_v.at[ci]], buf, sem).wait()
            pltpu.async_copy(buf, xs_hbm.at[dest_v.at[ci]], sem).wait()
            pltpu.async_copy(wbuf, ws_hbm.at[dest_v.at[ci]], sem).wait()

    return k(x, tok3, dest3, w16)


def _combine_sc(ye, ws, dest3, T):
    D = ye.shape[1]
    TOKC = CW // 2
    info = plsc.get_sparse_core_info()
    NC = info.num_cores
    mesh = plsc.VectorSubcoreMesh(core_axis_name="c", subcore_axis_name="s")

    @functools.partial(
        pl.kernel, mesh=mesh,
        out_type=jax.ShapeDtypeStruct((T, D), jnp.float32),
        scratch_types=[
            pltpu.VMEM((ACHUNK, CW), jnp.int32),
            pltpu.VMEM((CW, D), jnp.float32),
            pltpu.VMEM((CW, WPAD), jnp.float32),
            pltpu.VMEM((TOKC, D), jnp.float32),
            pltpu.SemaphoreType.DMA,
            pltpu.SemaphoreType.DMA,
        ],
    )
    def k(ye_hbm, ws_hbm, dest_hbm, out_hbm, dest_v, buf, wbuf, obuf,
          sem, sem2):
        wid = lax.axis_index("s") * NC + lax.axis_index("c")
        pltpu.sync_copy(dest_hbm.at[wid], dest_v)
        for ci in range(ACHUNK):
            cw = pltpu.async_copy(ws_hbm.at[dest_v.at[ci]], wbuf, sem2)
            pltpu.async_copy(ye_hbm.at[dest_v.at[ci]], buf, sem).wait()
            cw.wait()
            for j in range(TOKC):
                w0 = wbuf[2 * j, pl.ds(0, LANES)]
                w1 = wbuf[2 * j + 1, pl.ds(0, LANES)]

                def sbody(si, _, j=j, w0=w0, w1=w1):
                    sl = pl.ds(si * LANES, LANES)
                    obuf[j, sl] = w0 * buf[2 * j, sl] + w1 * buf[2 * j + 1, sl]
                    return 0

                lax.fori_loop(0, D // LANES, sbody, 0)
            pltpu.sync_copy(
                obuf, out_hbm.at[pl.ds(wid * (ACHUNK * TOKC) + ci * TOKC, TOKC)])

    return k(ye, ws, dest3)


def kernel(x, W_router, W_gate, W_up, W_down):
    T, D = x.shape
    E = W_gate.shape[0]
    tril = jnp.tril(jnp.ones((T, T), jnp.bfloat16), -1)
    dest, w, blkrange = _route(x, W_router, tril)
    tok3 = (jnp.arange(T * 2, dtype=jnp.int32) // 2).reshape(NW, ACHUNK, CW)
    dest3 = dest.reshape(NW, ACHUNK, CW)
    w16 = jnp.broadcast_to(w.reshape(-1)[:, None], (T * 2, WPAD))
    x_pk = lax.bitcast_convert_type(
        x.astype(jnp.bfloat16).reshape(T, D // 2, 2), jnp.int32)
    xs_pk, ws = _dispatch_sc(x_pk, tok3, dest3, w16)
    xs = lax.bitcast_convert_type(xs_pk, jnp.bfloat16).reshape(PADROWS, D)
    ye = _ffn(blkrange[0], blkrange[1], xs, W_gate, W_up, W_down)
    return _combine_sc(ye, ws, dest3, T)


# speedup vs baseline: 1.6721x; 1.6721x over previous
import functools

import jax
import jax.numpy as jnp
from jax import lax
from jax.experimental import pallas as pl
from jax.experimental.pallas import tpu as pltpu
from jax.experimental.pallas import tpu_sc as plsc

BLK = 256
MAXB = 24
PADROWS = MAXB * BLK
NW = 32
ACHUNK = 4
CW = 32
LANES = 16
WPAD = 128


def _route_body(x_ref, wr_ref, dest_ref, w_ref, blk_ref):
    T = x_ref.shape[0]
    E = wr_ref.shape[0]
    x = x_ref[...]
    rio = lax.broadcasted_iota(jnp.int32, (T, T), 0)
    cio = lax.broadcasted_iota(jnp.int32, (T, T), 1)
    tril = jnp.where(cio < rio, 1.0, 0.0).astype(jnp.bfloat16)
    logits = lax.dot_general(
        x, wr_ref[...], (((1,), (1,)), ((), ())),
        preferred_element_type=jnp.float32)
    eio = lax.broadcasted_iota(jnp.int32, (T, E), 1)
    m1 = jnp.max(logits, axis=-1, keepdims=True)
    idx1 = jnp.min(jnp.where(logits == m1, eio, E), axis=-1, keepdims=True)
    masked = jnp.where(eio == idx1, -1e30, logits)
    m2 = jnp.max(masked, axis=-1, keepdims=True)
    idx2 = jnp.min(jnp.where(masked == m2, eio, E), axis=-1, keepdims=True)
    e2 = jnp.exp(m2 - m1)
    w1 = 1.0 / (1.0 + e2)
    w2 = e2 / (1.0 + e2)
    sel = ((eio == idx1) | (eio == idx2)).astype(jnp.float32)

    pos = lax.dot_general(
        tril, sel.astype(jnp.bfloat16), (((1,), (0,)), ((), ())),
        preferred_element_type=jnp.float32)
    ones_row = jnp.ones((1, T), jnp.float32)
    counts = lax.dot_general(
        ones_row, sel, (((1,), (0,)), ((), ())),
        preferred_element_type=jnp.float32)
    padded = jnp.ceil(counts / BLK) * BLK
    r8 = lax.broadcasted_iota(jnp.int32, (E, E), 0)
    c8 = lax.broadcasted_iota(jnp.int32, (E, E), 1)
    upper = (r8 < c8).astype(jnp.float32)
    offs = lax.dot_general(
        padded, upper, (((1,), (0,)), ((), ())),
        preferred_element_type=jnp.float32)
    basep = offs + pos
    d1 = jnp.sum(jnp.where(eio == idx1, basep, 0.0), axis=-1, keepdims=True)
    d2 = jnp.sum(jnp.where(eio == idx2, basep, 0.0), axis=-1, keepdims=True)
    dest_ref[...] = jnp.concatenate([d1, d2], axis=1).astype(jnp.int32)
    w_ref[...] = jnp.concatenate(
        [jnp.broadcast_to(w1, (T, WPAD)), jnp.broadcast_to(w2, (T, WPAD))],
        axis=1)

    blk_ref[0:1, :] = (offs / BLK).astype(jnp.int32)
    blk_ref[1:2, :] = ((offs + padded) / BLK).astype(jnp.int32)


def _route(x, W_router):
    T = x.shape[0]
    E = W_router.shape[0]
    return pl.pallas_call(
        _route_body,
        out_shape=[
            jax.ShapeDtypeStruct((T, 2), jnp.int32),
            jax.ShapeDtypeStruct((T, 2 * WPAD), jnp.float32),
            jax.ShapeDtypeStruct((2, E), jnp.int32),
        ],
        compiler_params=pltpu.CompilerParams(
            vmem_limit_bytes=62 * 1024 * 1024),
    )(x, W_router)


NF = 8


def _ffn_body(st_ref, en_ref, xs_hbm, wg_ref, wu_ref, wd_ref,
              out_hbm, xs_v, out_v, sem, *, E):
    e = pl.program_id(0)
    f = pl.program_id(1)

    @pl.when((e == 0) & (f == 0))
    def _():
        pltpu.make_async_copy(xs_hbm, xs_v, sem).start()
        pltpu.make_async_copy(xs_hbm, xs_v, sem).wait()

    def body(b, carry):
        r0 = pl.multiple_of(b * BLK, BLK)
        sl = pl.ds(r0, BLK)
        xb = xs_v[sl, :]
        xg = lax.dot_general(
            xb, wg_ref[0], (((1,), (1,)), ((), ())),
            preferred_element_type=jnp.float32)
        xu = lax.dot_general(
            xb, wu_ref[0], (((1,), (1,)), ((), ())),
            preferred_element_type=jnp.float32)
        h = (xg / (1.0 + jnp.exp(-xg))) * xu
        part = lax.dot_general(
            h, wd_ref[0], (((1,), (1,)), ((), ())),
            preferred_element_type=jnp.float32)
        out_v[sl, :] = jnp.where(f == 0, part, out_v[sl, :] + part)
        return carry

    lax.fori_loop(st_ref[e], en_ref[e], body, 0)

    @pl.when((e == E - 1) & (f == NF - 1))
    def _():
        pltpu.make_async_copy(out_v, out_hbm, sem).start()
        pltpu.make_async_copy(out_v, out_hbm, sem).wait()


def _ffn(starts, ends, xs, Wg, Wu, Wd):
    E, F, D = Wg.shape
    FT = F // NF
    return pl.pallas_call(
        functools.partial(_ffn_body, E=E),
        grid=(E, NF),
        in_specs=[
            pl.BlockSpec(memory_space=pltpu.SMEM),
            pl.BlockSpec(memory_space=pltpu.SMEM),
            pl.BlockSpec(memory_space=pl.ANY),
            pl.BlockSpec((1, FT, D), lambda e, f: (e, f, 0)),
            pl.BlockSpec((1, FT, D), lambda e, f: (e, f, 0)),
            pl.BlockSpec((1, D, FT), lambda e, f: (e, 0, f)),
        ],
        out_specs=pl.BlockSpec(memory_space=pl.ANY),
        scratch_shapes=[
            pltpu.VMEM((PADROWS, D), jnp.float32),
            pltpu.VMEM((PADROWS, D), jnp.float32),
            pltpu.SemaphoreType.DMA,
        ],
        out_shape=jax.ShapeDtypeStruct((PADROWS, D), jnp.float32),
        compiler_params=pltpu.CompilerParams(
            vmem_limit_bytes=62 * 1024 * 1024),
    )(starts, ends, xs, Wg, Wu, Wd)


def _dispatch_sc(x, tok3, dest3, w16):
    T, D = x.shape
    APW = ACHUNK * CW
    info = plsc.get_sparse_core_info()
    NC = info.num_cores
    mesh = plsc.VectorSubcoreMesh(core_axis_name="c", subcore_axis_name="s")

    @functools.partial(
        pl.kernel, mesh=mesh,
        out_type=[
            jax.ShapeDtypeStruct((PADROWS, D), jnp.float32),
            jax.ShapeDtypeStruct((PADROWS, WPAD), jnp.float32),
        ],
        scratch_types=[
            pltpu.VMEM((ACHUNK, CW), jnp.int32),
            pltpu.VMEM((ACHUNK, CW), jnp.int32),
            pltpu.VMEM((CW, D), jnp.float32),
            pltpu.VMEM((CW, WPAD), jnp.float32),
            pltpu.SemaphoreType.DMA,
        ],
    )
    def k(x_hbm, tok_hbm, dest_hbm, w16_hbm, xs_hbm, ws_hbm,
          tok_v, dest_v, buf, wbuf, sem):
        wid = lax.axis_index("s") * NC + lax.axis_index("c")
        pltpu.sync_copy(tok_hbm.at[wid], tok_v)
        pltpu.sync_copy(dest_hbm.at[wid], dest_v)
        for ci in range(ACHUNK):
            pltpu.sync_copy(w16_hbm.at[pl.ds(wid * APW + ci * CW, CW)], wbuf)
            pltpu.async_copy(x_hbm.at[tok_v.at[ci]], buf, sem).wait()
            pltpu.async_copy(buf, xs_hbm.at[dest_v.at[ci]], sem).wait()
            pltpu.async_copy(wbuf, ws_hbm.at[dest_v.at[ci]], sem).wait()

    return k(x, tok3, dest3, w16)


def _combine_sc(ye, ws, dest3, T):
    D = ye.shape[1]
    TOKC = CW // 2
    info = plsc.get_sparse_core_info()
    NC = info.num_cores
    mesh = plsc.VectorSubcoreMesh(core_axis_name="c", subcore_axis_name="s")

    @functools.partial(
        pl.kernel, mesh=mesh,
        out_type=jax.ShapeDtypeStruct((T, D), jnp.float32),
        scratch_types=[
            pltpu.VMEM((ACHUNK, CW), jnp.int32),
            pltpu.VMEM((CW, D), jnp.float32),
            pltpu.VMEM((CW, WPAD), jnp.float32),
            pltpu.VMEM((TOKC, D), jnp.float32),
            pltpu.SemaphoreType.DMA,
            pltpu.SemaphoreType.DMA,
        ],
    )
    def k(ye_hbm, ws_hbm, dest_hbm, out_hbm, dest_v, buf, wbuf, obuf,
          sem, sem2):
        wid = lax.axis_index("s") * NC + lax.axis_index("c")
        pltpu.sync_copy(dest_hbm.at[wid], dest_v)
        for ci in range(ACHUNK):
            cw = pltpu.async_copy(ws_hbm.at[dest_v.at[ci]], wbuf, sem2)
            pltpu.async_copy(ye_hbm.at[dest_v.at[ci]], buf, sem).wait()
            cw.wait()
            for j in range(TOKC):
                w0 = wbuf[2 * j, pl.ds(0, LANES)]
                w1 = wbuf[2 * j + 1, pl.ds(0, LANES)]

                def sbody(si, _, j=j, w0=w0, w1=w1):
                    sl = pl.ds(si * LANES, LANES)
                    obuf[j, sl] = w0 * buf[2 * j, sl] + w1 * buf[2 * j + 1, sl]
                    return 0

                lax.fori_loop(0, D // LANES, sbody, 0)
            pltpu.sync_copy(
                obuf, out_hbm.at[pl.ds(wid * (ACHUNK * TOKC) + ci * TOKC, TOKC)])

    return k(ye, ws, dest3)


def kernel(x, W_router, W_gate, W_up, W_down):
    T, D = x.shape
    E = W_gate.shape[0]
    dest, w, blkrange = _route(x, W_router)
    tok3 = (jnp.arange(T * 2, dtype=jnp.int32) // 2).reshape(NW, ACHUNK, CW)
    dest3 = dest.reshape(NW, ACHUNK, CW)
    w16 = w.reshape(T * 2, WPAD)
    xs, ws = _dispatch_sc(x, tok3, dest3, w16)
    ye = _ffn(blkrange[0], blkrange[1], xs, W_gate, W_up, W_down)
    return _combine_sc(ye, ws, dest3, T)


# speedup vs baseline: 1.9333x; 1.1562x over previous
import functools

import jax
import jax.numpy as jnp
from jax import lax
from jax.experimental import pallas as pl
from jax.experimental.pallas import tpu as pltpu
from jax.experimental.pallas import tpu_sc as plsc

BLK = 256
MAXB = 24
PADROWS = MAXB * BLK
NW = 32
ACHUNK = 4
CW = 32
LANES = 16
WPAD = 128


def _route_body(x_ref, wr_ref, dest_ref, w_ref, blk_ref):
    T = x_ref.shape[0]
    E = wr_ref.shape[0]
    x = x_ref[...]
    rio = lax.broadcasted_iota(jnp.int32, (T, T), 0)
    cio = lax.broadcasted_iota(jnp.int32, (T, T), 1)
    tril = jnp.where(cio < rio, 1.0, 0.0).astype(jnp.bfloat16)
    logits = lax.dot_general(
        x, wr_ref[...], (((1,), (1,)), ((), ())),
        preferred_element_type=jnp.float32)
    eio = lax.broadcasted_iota(jnp.int32, (T, E), 1)
    m1 = jnp.max(logits, axis=-1, keepdims=True)
    idx1 = jnp.min(jnp.where(logits == m1, eio, E), axis=-1, keepdims=True)
    masked = jnp.where(eio == idx1, -1e30, logits)
    m2 = jnp.max(masked, axis=-1, keepdims=True)
    idx2 = jnp.min(jnp.where(masked == m2, eio, E), axis=-1, keepdims=True)
    e2 = jnp.exp(m2 - m1)
    w1 = 1.0 / (1.0 + e2)
    w2 = e2 / (1.0 + e2)
    sel = ((eio == idx1) | (eio == idx2)).astype(jnp.float32)

    pos = lax.dot_general(
        tril, sel.astype(jnp.bfloat16), (((1,), (0,)), ((), ())),
        preferred_element_type=jnp.float32)
    ones_row = jnp.ones((1, T), jnp.float32)
    counts = lax.dot_general(
        ones_row, sel, (((1,), (0,)), ((), ())),
        preferred_element_type=jnp.float32)
    padded = jnp.ceil(counts / BLK) * BLK
    r8 = lax.broadcasted_iota(jnp.int32, (E, E), 0)
    c8 = lax.broadcasted_iota(jnp.int32, (E, E), 1)
    upper = (r8 < c8).astype(jnp.float32)
    offs = lax.dot_general(
        padded, upper, (((1,), (0,)), ((), ())),
        preferred_element_type=jnp.float32)
    basep = offs + pos
    d1 = jnp.sum(jnp.where(eio == idx1, basep, 0.0), axis=-1, keepdims=True)
    d2 = jnp.sum(jnp.where(eio == idx2, basep, 0.0), axis=-1, keepdims=True)
    dest_ref[...] = jnp.concatenate([d1, d2], axis=1).astype(jnp.int32)
    w_ref[...] = jnp.concatenate(
        [jnp.broadcast_to(w1, (T, WPAD)), jnp.broadcast_to(w2, (T, WPAD))],
        axis=1)

    blk_ref[0:1, :] = (offs / BLK).astype(jnp.int32)
    blk_ref[1:2, :] = ((offs + padded) / BLK).astype(jnp.int32)


def _route(x, W_router):
    T = x.shape[0]
    E = W_router.shape[0]
    return pl.pallas_call(
        _route_body,
        out_shape=[
            jax.ShapeDtypeStruct((T, 2), jnp.int32),
            jax.ShapeDtypeStruct((T, 2 * WPAD), jnp.float32),
            jax.ShapeDtypeStruct((2, E), jnp.int32),
        ],
        compiler_params=pltpu.CompilerParams(
            vmem_limit_bytes=62 * 1024 * 1024),
    )(x, W_router)


NF = 4


def _ffn_body(st_ref, en_ref, xs_hbm, wg_ref, wu_ref, wd_ref,
              out_hbm, xs_v, out_v, sem, *, E):
    e = pl.program_id(0)
    f = pl.program_id(1)

    @pl.when((e == 0) & (f == 0))
    def _():
        pltpu.make_async_copy(xs_hbm, xs_v, sem).start()
        pltpu.make_async_copy(xs_hbm, xs_v, sem).wait()

    def body(b, carry):
        r0 = pl.multiple_of(b * BLK, BLK)
        sl = pl.ds(r0, BLK)
        xb = xs_v[sl, :]
        xg = lax.dot_general(
            xb, wg_ref[0], (((1,), (1,)), ((), ())),
            preferred_element_type=jnp.float32)
        xu = lax.dot_general(
            xb, wu_ref[0], (((1,), (1,)), ((), ())),
            preferred_element_type=jnp.float32)
        h = (xg / (1.0 + jnp.exp(-xg))) * xu
        part = lax.dot_general(
            h, wd_ref[0], (((1,), (1,)), ((), ())),
            preferred_element_type=jnp.float32)
        out_v[sl, :] = jnp.where(f == 0, part, out_v[sl, :] + part)
        return carry

    lax.fori_loop(st_ref[e], en_ref[e], body, 0)

    @pl.when((e == E - 1) & (f == NF - 1))
    def _():
        pltpu.make_async_copy(out_v, out_hbm, sem).start()
        pltpu.make_async_copy(out_v, out_hbm, sem).wait()


def _ffn(starts, ends, xs, Wg, Wu, Wd):
    E, F, D = Wg.shape
    FT = F // NF
    return pl.pallas_call(
        functools.partial(_ffn_body, E=E),
        grid=(E, NF),
        in_specs=[
            pl.BlockSpec(memory_space=pltpu.SMEM),
            pl.BlockSpec(memory_space=pltpu.SMEM),
            pl.BlockSpec(memory_space=pl.ANY),
            pl.BlockSpec((1, FT, D), lambda e, f: (e, f, 0)),
            pl.BlockSpec((1, FT, D), lambda e, f: (e, f, 0)),
            pl.BlockSpec((1, D, FT), lambda e, f: (e, 0, f)),
        ],
        out_specs=pl.BlockSpec(memory_space=pl.ANY),
        scratch_shapes=[
            pltpu.VMEM((PADROWS, D), jnp.float32),
            pltpu.VMEM((PADROWS, D), jnp.float32),
            pltpu.SemaphoreType.DMA,
        ],
        out_shape=jax.ShapeDtypeStruct((PADROWS, D), jnp.float32),
        compiler_params=pltpu.CompilerParams(
            vmem_limit_bytes=62 * 1024 * 1024),
    )(starts, ends, xs, Wg, Wu, Wd)


def _dispatch_sc(x, tok3, dest3, w16):
    T, D = x.shape
    APW = ACHUNK * CW
    info = plsc.get_sparse_core_info()
    NC = info.num_cores
    mesh = plsc.VectorSubcoreMesh(core_axis_name="c", subcore_axis_name="s")

    @functools.partial(
        pl.kernel, mesh=mesh,
        out_type=[
            jax.ShapeDtypeStruct((PADROWS, D), jnp.float32),
            jax.ShapeDtypeStruct((PADROWS, WPAD), jnp.float32),
        ],
        scratch_types=[
            pltpu.VMEM((ACHUNK, CW), jnp.int32),
            pltpu.VMEM((ACHUNK, CW), jnp.int32),
            pltpu.VMEM((CW, D), jnp.float32),
            pltpu.VMEM((CW, WPAD), jnp.float32),
            pltpu.SemaphoreType.DMA,
        ],
    )
    def k(x_hbm, tok_hbm, dest_hbm, w16_hbm, xs_hbm, ws_hbm,
          tok_v, dest_v, buf, wbuf, sem):
        wid = lax.axis_index("s") * NC + lax.axis_index("c")
        pltpu.sync_copy(tok_hbm.at[wid], tok_v)
        pltpu.sync_copy(dest_hbm.at[wid], dest_v)
        for ci in range(ACHUNK):
            pltpu.sync_copy(w16_hbm.at[pl.ds(wid * APW + ci * CW, CW)], wbuf)
            pltpu.async_copy(x_hbm.at[tok_v.at[ci]], buf, sem).wait()
            pltpu.async_copy(buf, xs_hbm.at[dest_v.at[ci]], sem).wait()
            pltpu.async_copy(wbuf, ws_hbm.at[dest_v.at[ci]], sem).wait()

    return k(x, tok3, dest3, w16)


def _combine_sc(ye, ws, dest3, T):
    D = ye.shape[1]
    TOKC = CW // 2
    info = plsc.get_sparse_core_info()
    NC = info.num_cores
    mesh = plsc.VectorSubcoreMesh(core_axis_name="c", subcore_axis_name="s")

    @functools.partial(
        pl.kernel, mesh=mesh,
        out_type=jax.ShapeDtypeStruct((T, D), jnp.float32),
        scratch_types=[
            pltpu.VMEM((ACHUNK, CW), jnp.int32),
            pltpu.VMEM((CW, D), jnp.float32),
            pltpu.VMEM((CW, WPAD), jnp.float32),
            pltpu.VMEM((TOKC, D), jnp.float32),
            pltpu.SemaphoreType.DMA,
            pltpu.SemaphoreType.DMA,
        ],
    )
    def k(ye_hbm, ws_hbm, dest_hbm, out_hbm, dest_v, buf, wbuf, obuf,
          sem, sem2):
        wid = lax.axis_index("s") * NC + lax.axis_index("c")
        pltpu.sync_copy(dest_hbm.at[wid], dest_v)
        for ci in range(ACHUNK):
            cw = pltpu.async_copy(ws_hbm.at[dest_v.at[ci]], wbuf, sem2)
            pltpu.async_copy(ye_hbm.at[dest_v.at[ci]], buf, sem).wait()
            cw.wait()
            for j in range(TOKC):
                w0 = wbuf[2 * j, pl.ds(0, LANES)]
                w1 = wbuf[2 * j + 1, pl.ds(0, LANES)]

                def sbody(si, _, j=j, w0=w0, w1=w1):
                    sl = pl.ds(si * LANES, LANES)
                    obuf[j, sl] = w0 * buf[2 * j, sl] + w1 * buf[2 * j + 1, sl]
                    return 0

                lax.fori_loop(0, D // LANES, sbody, 0)
            pltpu.sync_copy(
                obuf, out_hbm.at[pl.ds(wid * (ACHUNK * TOKC) + ci * TOKC, TOKC)])

    return k(ye, ws, dest3)


def kernel(x, W_router, W_gate, W_up, W_down):
    T, D = x.shape
    E = W_gate.shape[0]
    dest, w, blkrange = _route(x, W_router)
    tok3 = (jnp.arange(T * 2, dtype=jnp.int32) // 2).reshape(NW, ACHUNK, CW)
    dest3 = dest.reshape(NW, ACHUNK, CW)
    w16 = w.reshape(T * 2, WPAD)
    xs, ws = _dispatch_sc(x, tok3, dest3, w16)
    ye = _ffn(blkrange[0], blkrange[1], xs, W_gate, W_up, W_down)
    return _combine_sc(ye, ws, dest3, T)
